# Initial kernel scaffold; baseline (speedup 1.0000x reference)
import jax
import jax.numpy as jnp
from jax.experimental import pallas as pl


def kernel(x, edge_index, batch, edge_attr, We, bE, W1, b1, g1, beta1, W2, b2, t, ng, nb):
    raise NotImplementedError("write your pallas kernel here")


# speedup vs baseline: 2.1752x; 2.1752x over previous
import functools

import jax
import jax.numpy as jnp
from jax import lax
from jax.experimental import pallas as pl
from jax.experimental.pallas import tpu as pltpu
from jax.experimental.pallas import tpu_sc as plsc

N = 10000
E = 320000
D = 128
DH = 256
L = 7
ED = 4
EPS = 1e-7

NC = 2
NS = 16
NWORK = NC * NS
NPW = (N + NWORK - 1) // NWORK
NP = NWORK * NPW
CHUNK = 128


def _sc_edge_body(cur_h, src_h, dst_h, ea_h, bounds_h, we_h, be_h, t_h,
                  num_h, den_h,
                  idx_v, dst_v, ea_v, xg_v, num_v, den_v, we_v, be_v, t_v,
                  bounds_v, sem):
    wid = lax.axis_index("s") * NC + lax.axis_index("c")
    pltpu.sync_copy(bounds_h, bounds_v)
    pltpu.sync_copy(we_h, we_v)
    pltpu.sync_copy(be_h, be_v)
    pltpu.sync_copy(t_h, t_v)
    iota = lax.iota(jnp.int32, 16)
    widv = jnp.full((16,), wid, jnp.int32)
    estart = jnp.max(plsc.load_gather(bounds_v, [widv]))
    eend = jnp.max(plsc.load_gather(bounds_v, [widv + NWORK]))
    nstart = wid * NPW
    tv = t_v[...]

    zero16 = jnp.zeros((16,), jnp.float32)

    def zbody(i, carry):
        zidx = jnp.full((16,), i * 16, jnp.int32) + iota
        plsc.store_scatter(num_v, [zidx], zero16)
        plsc.store_scatter(den_v, [zidx], zero16)
        return carry

    lax.fori_loop(0, NPW * D // 16, zbody, 0)

    c0 = estart // CHUNK
    c1 = (eend + CHUNK - 1) // CHUNK

    def chunk_body(c, carry):
        base = c * CHUNK
        pltpu.sync_copy(src_h.at[pl.ds(base, CHUNK)], idx_v)
        pltpu.sync_copy(dst_h.at[pl.ds(base, CHUNK)], dst_v)
        pltpu.sync_copy(ea_h.at[pl.ds(base * ED, CHUNK * ED)], ea_v)
        pltpu.async_copy(cur_h.at[idx_v], xg_v, sem).wait()
        j0 = jnp.maximum(estart - base, 0)
        j1 = jnp.minimum(eend - base, CHUNK)

        def edge_body(j, ecarry):
            jv = jnp.full((16,), j, jnp.int32)
            dvec = plsc.load_gather(dst_v, [jv]) - nstart
            a0 = plsc.load_gather(ea_v, [jv * ED])
            a1 = plsc.load_gather(ea_v, [jv * ED + 1])
            a2 = plsc.load_gather(ea_v, [jv * ED + 2])
            a3 = plsc.load_gather(ea_v, [jv * ED + 3])
            dbase = dvec * D + iota
            for c8 in range(D // 16):
                off = c8 * 16
                xv = plsc.load_gather(xg_v, [jv, iota + off])
                ev = (be_v[pl.ds(off, 16)]
                      + a0 * we_v[pl.ds(off, 16)]
                      + a1 * we_v[pl.ds(D + off, 16)]
                      + a2 * we_v[pl.ds(2 * D + off, 16)]
                      + a3 * we_v[pl.ds(3 * D + off, 16)])
                msg = jnp.maximum(xv + ev, 0.0) + jnp.float32(EPS)
                ex = jnp.exp(tv * msg)
                plsc.addupdate_scatter(num_v, [dbase + off], msg * ex)
                plsc.addupdate_scatter(den_v, [dbase + off], ex)
            return ecarry

        lax.fori_loop(j0, j1, edge_body, 0)
        return carry

    lax.fori_loop(c0, c1, chunk_body, 0)
    pltpu.sync_copy(num_v, num_h.at[pl.ds(nstart * D, NPW * D)])
    pltpu.sync_copy(den_v, den_h.at[pl.ds(nstart * D, NPW * D)])


@functools.lru_cache(maxsize=None)
def _get_sc_edge():
    return functools.partial(
        pl.kernel,
        mesh=plsc.VectorSubcoreMesh(core_axis_name="c", subcore_axis_name="s"),
        out_type=(jax.ShapeDtypeStruct((NP * D,), jnp.float32),
                  jax.ShapeDtypeStruct((NP * D,), jnp.float32)),
        scratch_types=[
            pltpu.VMEM((CHUNK,), jnp.int32),
            pltpu.VMEM((CHUNK,), jnp.int32),
            pltpu.VMEM((CHUNK * ED,), jnp.float32),
            pltpu.VMEM((CHUNK, D), jnp.float32),
            pltpu.VMEM((NPW * D,), jnp.float32),
            pltpu.VMEM((NPW * D,), jnp.float32),
            pltpu.VMEM((ED * D,), jnp.float32),
            pltpu.VMEM((D,), jnp.float32),
            pltpu.VMEM((16,), jnp.float32),
            pltpu.VMEM((128,), jnp.int32),
            pltpu.SemaphoreType.DMA,
        ],
        compiler_params=pltpu.CompilerParams(needs_layout_passes=False),
    )(_sc_edge_body)


def _tc_mid_body(num_ref, den_ref, cur_ref, w1_ref, b1_ref, mid_ref):
    out = num_ref[...] / (den_ref[...] + 1e-16) + cur_ref[...]
    mid = jnp.dot(out, w1_ref[...], preferred_element_type=jnp.float32)
    mid_ref[...] = mid + b1_ref[...]


_tc_mid = pl.pallas_call(
    _tc_mid_body,
    out_shape=jax.ShapeDtypeStruct((N, DH), jnp.float32),
)


def _tc_h_body(mid_ref, mu_ref, var_ref, g1_ref, bt_ref, w2_ref, b2_ref,
               h_ref, hout_ref):
    midn = (g1_ref[...] * (mid_ref[...] - mu_ref[...])
            / jnp.sqrt(var_ref[...] + 1e-5) + bt_ref[...])
    midn = jnp.maximum(midn, 0.0)
    part = jnp.dot(midn, w2_ref[...], preferred_element_type=jnp.float32)
    hout_ref[...] = h_ref[...] + (part + b2_ref[...])


_tc_h = pl.pallas_call(
    _tc_h_body,
    out_shape=jax.ShapeDtypeStruct((N, D), jnp.float32),
)


def _tc_r_body(h_ref, mu_ref, var_ref, ng_ref, nb_ref, rout_ref):
    hn = (ng_ref[...] * (h_ref[...] - mu_ref[...])
          / jnp.sqrt(var_ref[...] + 1e-5) + nb_ref[...])
    rout_ref[...] = jnp.where(hn >= 0, hn, 0.01 * hn)


_tc_r = pl.pallas_call(
    _tc_r_body,
    out_shape=jax.ShapeDtypeStruct((N, D), jnp.float32),
)


def kernel(x, edge_index, batch, edge_attr, We, bE, W1, b1, g1, beta1, W2,
           b2, t, ng, nb):
    src = edge_index[0]
    dst = edge_index[1]
    sorted_ops = jax.lax.sort(
        (dst, src, edge_attr[:, 0], edge_attr[:, 1], edge_attr[:, 2],
         edge_attr[:, 3]), num_keys=1, is_stable=True)
    dsts, srcs = sorted_ops[0], sorted_ops[1]
    eap = (jnp.stack(sorted_ops[2:], axis=1)
           .astype(jnp.bfloat16).astype(jnp.float32).reshape(-1))
    We_r = We.astype(jnp.bfloat16).astype(jnp.float32)
    node_bounds = jnp.minimum(jnp.arange(NWORK + 1, dtype=jnp.int32) * NPW, N)
    offs = jnp.searchsorted(dsts, node_bounds).astype(jnp.int32)
    bounds = jnp.concatenate(
        [offs[:NWORK], offs[1:], jnp.zeros((128 - 2 * NWORK,), jnp.int32)])

    h = jnp.zeros((N, D), jnp.float32)
    cur = x
    r = x
    for l in range(L):
        t16 = jnp.broadcast_to(t[l], (16,))
        num_f, den_f = _get_sc_edge()(cur, srcs, dsts, eap, bounds,
                                      We_r[l].reshape(-1), bE[l], t16)
        num2 = num_f.reshape(NP, D)[:N]
        den2 = den_f.reshape(NP, D)[:N]
        ng_n, nb_n = (ng[l + 1], nb[l + 1]) if l + 1 < L else (ng[0], nb[0])
        mid = _tc_mid(num2, den2, cur, W1[l], b1[l].reshape(1, DH))
        mu = jnp.mean(mid, axis=0)
        var = jnp.var(mid, axis=0)
        h = _tc_h(mid, mu.reshape(1, DH), var.reshape(1, DH),
                  g1[l].reshape(1, DH), beta1[l].reshape(1, DH), W2[l],
                  b2[l].reshape(1, D), h)
        hmu = jnp.mean(h, axis=0)
        hvar = jnp.var(h, axis=0)
        r = _tc_r(h, hmu.reshape(1, D), hvar.reshape(1, D),
                  ng_n.reshape(1, D), nb_n.reshape(1, D))
        cur = r
    return r
